# Initial kernel scaffold; baseline (speedup 1.0000x reference)
import jax
import jax.numpy as jnp
from jax.experimental import pallas as pl


def kernel(x, W_word, W_bigram, W_field, fc1_w, fc1_b, fc2_w, fc2_b):
    raise NotImplementedError("write your pallas kernel here")


# speedup vs baseline: 2.2938x; 2.2938x over previous
import functools

import jax
import jax.numpy as jnp
from jax import lax
from jax.experimental import pallas as pl
from jax.experimental.pallas import tpu as pltpu
from jax.experimental.pallas import tpu_sc as plsc

B = 4096
L = 200
EMBED = 32
HID = 128
CLASSES = 10

_info = plsc.get_sparse_core_info()
NC = _info.num_cores
NS = _info.num_subcores
LANES = _info.num_lanes
NW = NC * NS
BPW = B // NW
CB = 2
NCHUNK = BPW // CB
ROWS = CB * L
G = 80
NG = ROWS // G


def _sc_pool(x2, w_word, w_bigram, w_field):
    mesh = plsc.VectorSubcoreMesh(core_axis_name="c", subcore_axis_name="s")

    @functools.partial(
        pl.kernel,
        mesh=mesh,
        out_type=jax.ShapeDtypeStruct((B * 2 * EMBED,), jnp.float32),
        scratch_types=[
            pltpu.VMEM((3 * ROWS,), jnp.int32),
            pltpu.VMEM((3 * ROWS, EMBED), jnp.float32),
            pltpu.VMEM((CB * 2 * EMBED,), jnp.float32),
            pltpu.SemaphoreType.DMA,
        ],
        compiler_params=pltpu.CompilerParams(use_tc_tiling_on_sc=False),
    )
    def body(x2_hbm, ww_hbm, wb_hbm, wf_hbm, out_hbm, idx_v, rows_v, ost_v, sem):
        wid = lax.axis_index("s") * NC + lax.axis_index("c")
        tabs = (ww_hbm, wb_hbm, wf_hbm)

        def chunk_body(c, _):
            row0 = pl.multiple_of(wid * BPW + c * CB, CB)
            off0 = pl.multiple_of(row0 * L, ROWS)
            for t in range(3):
                pltpu.sync_copy(
                    x2_hbm.at[pl.ds(t * (B * L) + off0, ROWS)],
                    idx_v.at[pl.ds(t * ROWS, ROWS)])
            handles = []
            for t in range(3):
                for j in range(NG):
                    handles.append(pltpu.async_copy(
                        tabs[t].at[idx_v.at[pl.ds(t * ROWS + j * G, G)]],
                        rows_v.at[pl.ds(t * ROWS + j * G, G)],
                        sem,
                    ))
            for h in handles:
                h.wait()
            for b in range(CB):
                def acc_body(l, carry):
                    a0, a1, f0, f1, g0, g1 = carry
                    r = b * L + l
                    a0 = a0 + rows_v[r, pl.ds(0, LANES)]
                    a1 = a1 + rows_v[r, pl.ds(LANES, LANES)]
                    g0 = g0 + rows_v[ROWS + r, pl.ds(0, LANES)]
                    g1 = g1 + rows_v[ROWS + r, pl.ds(LANES, LANES)]
                    f0 = f0 + rows_v[2 * ROWS + r, pl.ds(0, LANES)]
                    f1 = f1 + rows_v[2 * ROWS + r, pl.ds(LANES, LANES)]
                    return a0, a1, f0, f1, g0, g1
                z = jnp.zeros((LANES,), jnp.float32)
                a0, a1, f0, f1, g0, g1 = lax.fori_loop(
                    0, L, acc_body, (z, z, z, z, z, z))
                inv = jnp.float32(1.0 / L)
                o = b * 2 * EMBED
                ost_v[pl.ds(o, LANES)] = (a0 + 5.0 * f0) * inv
                ost_v[pl.ds(o + LANES, LANES)] = (a1 + 5.0 * f1) * inv
                ost_v[pl.ds(o + 2 * LANES, LANES)] = g0 * inv
                ost_v[pl.ds(o + 3 * LANES, LANES)] = g1 * inv
            pltpu.sync_copy(
                ost_v, out_hbm.at[pl.ds(row0 * 2 * EMBED, CB * 2 * EMBED)])
            return 0

        lax.fori_loop(0, NCHUNK, chunk_body, 0)

    return body(x2, w_word, w_bigram, w_field)


def _mlp_body(p_ref, w1_ref, b1_ref, w2_ref, b2_ref, o_ref):
    h = jnp.dot(p_ref[...], w1_ref[...], preferred_element_type=jnp.float32)
    h = jnp.maximum(h + b1_ref[...], 0.0)
    o_ref[...] = (
        jnp.dot(h, w2_ref[...], preferred_element_type=jnp.float32)
        + b2_ref[...]
    )


def _mlp(pooled, fc1_w, fc1_b, fc2_w, fc2_b):
    w2p = jnp.pad(fc2_w, ((0, 0), (0, HID - CLASSES)))
    b2p = jnp.pad(fc2_b, (0, HID - CLASSES)).reshape(1, HID)
    out = pl.pallas_call(
        _mlp_body,
        out_shape=jax.ShapeDtypeStruct((B, HID), jnp.float32),
    )(pooled, fc1_w, fc1_b.reshape(1, HID), w2p, b2p)
    return out[:, :CLASSES]


def kernel(x, W_word, W_bigram, W_field, fc1_w, fc1_b, fc2_w, fc2_b):
    x2 = x.reshape(3 * B * L).astype(jnp.int32)
    pooled = _sc_pool(x2, W_word, W_bigram, W_field).reshape(B, 2 * EMBED)
    return _mlp(pooled, fc1_w, fc1_b, fc2_w, fc2_b)
